# Initial kernel scaffold; baseline (speedup 1.0000x reference)
import jax
import jax.numpy as jnp
from jax.experimental import pallas as pl


def kernel(plan_ids, weight):
    raise NotImplementedError("write your pallas kernel here")


# speedup vs baseline: 1.8746x; 1.8746x over previous
import functools

import jax
import jax.numpy as jnp
from jax import lax
from jax.experimental import pallas as pl
from jax.experimental.pallas import tpu as pltpu
from jax.experimental.pallas import tpu_sc as plsc

NC = 2
NS = 16
NW = NC * NS

VOCAB = 1_000_000
D = 64
B = 16384 * 50
BP = B // NW
CHUNK = 128
C = BP // CHUNK
NBUF = 8
GROUPS = C // NBUF


@functools.partial(
    pl.kernel,
    mesh=plsc.VectorSubcoreMesh(core_axis_name="c", subcore_axis_name="s"),
    out_type=jax.ShapeDtypeStruct((B, D), jnp.float32),
    scratch_types=[
        pltpu.VMEM((C, CHUNK), jnp.int32),
        pltpu.VMEM((NBUF, CHUNK, D), jnp.float32),
        pltpu.SemaphoreType.DMA,
        pltpu.SemaphoreType.DMA,
    ],
    compiler_params=pltpu.CompilerParams(use_tc_tiling_on_sc=False),
)
def _gather_kernel(table_hbm, idx_hbm, out_hbm, idx_v, bufs, gsem, wsem):
    wid = lax.axis_index("s") * NC + lax.axis_index("c")
    row0 = wid * BP

    pltpu.sync_copy(idx_hbm.at[wid], idx_v)

    def group(g, carry):
        for b in range(NBUF):
            c = g * NBUF + b
            pltpu.async_copy(table_hbm.at[idx_v.at[c]], bufs.at[b], gsem)
        for b in range(NBUF):
            c = g * NBUF + b
            pltpu.make_async_copy(table_hbm.at[idx_v.at[0]], bufs.at[b], gsem).wait()
            pltpu.async_copy(
                bufs.at[b],
                out_hbm.at[pl.ds(row0 + c * CHUNK, CHUNK)],
                wsem,
            )
        for b in range(NBUF):
            pltpu.make_async_copy(
                bufs.at[b], out_hbm.at[pl.ds(row0, CHUNK)], wsem
            ).wait()
        return carry

    lax.fori_loop(0, GROUPS, group, 0)


def kernel(plan_ids, weight):
    idx = plan_ids.reshape(NW, C, CHUNK).astype(jnp.int32)
    out = _gather_kernel(weight, idx)
    return out.reshape(plan_ids.shape[0], plan_ids.shape[1], D)


# speedup vs baseline: 1.8763x; 1.0009x over previous
import functools

import jax
import jax.numpy as jnp
from jax import lax
from jax.experimental import pallas as pl
from jax.experimental.pallas import tpu as pltpu
from jax.experimental.pallas import tpu_sc as plsc

NC = 2
NS = 16
NW = NC * NS

VOCAB = 1_000_000
D = 64
B = 16384 * 50
BP = B // NW
CHUNK = 128
C = BP // CHUNK
NBUF = 4
GROUP = NBUF * CHUNK
G = C // NBUF
R = 2
STEADY = (G - R) // R


@functools.partial(
    pl.kernel,
    mesh=plsc.VectorSubcoreMesh(core_axis_name="c", subcore_axis_name="s"),
    out_type=jax.ShapeDtypeStruct((B, D), jnp.float32),
    scratch_types=[
        pltpu.VMEM((C, CHUNK), jnp.int32),
        pltpu.VMEM((R, GROUP, D), jnp.float32),
        pltpu.SemaphoreType.DMA,
        pltpu.SemaphoreType.DMA,
    ],
    compiler_params=pltpu.CompilerParams(use_tc_tiling_on_sc=False),
)
def _gather_kernel(table_hbm, idx_hbm, out_hbm, idx_v, bufs, gsem, wsem):
    wid = lax.axis_index("s") * NC + lax.axis_index("c")
    row0 = wid * BP

    pltpu.sync_copy(idx_hbm.at[wid], idx_v)

    def fire_gathers(g, r):
        for b in range(NBUF):
            pltpu.async_copy(
                table_hbm.at[idx_v.at[g * NBUF + b]],
                bufs.at[r, pl.ds(b * CHUNK, CHUNK)],
                gsem,
            )

    def wait_gathers(r):
        for b in range(NBUF):
            pltpu.make_async_copy(
                table_hbm.at[idx_v.at[0]],
                bufs.at[r, pl.ds(b * CHUNK, CHUNK)],
                gsem,
            ).wait()

    def fire_write(g, r):
        pltpu.async_copy(
            bufs.at[r], out_hbm.at[pl.ds(row0 + g * GROUP, GROUP)], wsem
        )

    def wait_write(r):
        pltpu.make_async_copy(
            bufs.at[r], out_hbm.at[pl.ds(row0, GROUP)], wsem
        ).wait()

    for r in range(R):
        fire_gathers(r, r)

    def body(i, carry):
        g0 = i * R
        for r in range(R):
            wait_gathers(r)
            fire_write(g0 + r, r)
        for r in range(R):
            wait_write(r)
            fire_gathers(g0 + R + r, r)
        return carry

    lax.fori_loop(0, STEADY, body, 0)

    g0 = STEADY * R
    for r in range(R):
        wait_gathers(r)
        fire_write(g0 + r, r)
    for r in range(R):
        wait_write(r)


def kernel(plan_ids, weight):
    idx = plan_ids.reshape(NW, C, CHUNK).astype(jnp.int32)
    out = _gather_kernel(weight, idx)
    return out.reshape(plan_ids.shape[0], plan_ids.shape[1], D)


# speedup vs baseline: 1.8857x; 1.0050x over previous
import functools

import jax
import jax.numpy as jnp
from jax import lax
from jax.experimental import pallas as pl
from jax.experimental.pallas import tpu as pltpu
from jax.experimental.pallas import tpu_sc as plsc

NC = 2
NS = 16
NW = NC * NS

VOCAB = 1_000_000
D = 64
N = 16384
K = 50
NP = N // NW
NBUF = 8
G = NP // NBUF
R = 2
STEADY = (G - R) // R


@functools.partial(
    pl.kernel,
    mesh=plsc.VectorSubcoreMesh(core_axis_name="c", subcore_axis_name="s"),
    out_type=jax.ShapeDtypeStruct((N, K, D), jnp.float32),
    scratch_types=[
        pltpu.VMEM((NP, K), jnp.int32),
        pltpu.VMEM((R, NBUF, K, D), jnp.float32),
        pltpu.SemaphoreType.DMA,
        pltpu.SemaphoreType.DMA,
    ],
    compiler_params=pltpu.CompilerParams(use_tc_tiling_on_sc=False),
)
def _gather_kernel(table_hbm, idx_hbm, out_hbm, idx_v, bufs, gsem, wsem):
    wid = lax.axis_index("s") * NC + lax.axis_index("c")
    i0 = wid * NP

    pltpu.sync_copy(idx_hbm.at[pl.ds(i0, NP)], idx_v)

    def fire_gathers(g, r):
        for b in range(NBUF):
            pltpu.async_copy(
                table_hbm.at[idx_v.at[g * NBUF + b]],
                bufs.at[r, b],
                gsem,
            )

    def wait_gathers(r):
        for b in range(NBUF):
            pltpu.make_async_copy(
                table_hbm.at[idx_v.at[0]], bufs.at[r, b], gsem
            ).wait()

    def fire_write(g, r):
        pltpu.async_copy(
            bufs.at[r], out_hbm.at[pl.ds(i0 + g * NBUF, NBUF)], wsem
        )

    def wait_write(r):
        pltpu.make_async_copy(
            bufs.at[r], out_hbm.at[pl.ds(i0, NBUF)], wsem
        ).wait()

    for r in range(R):
        fire_gathers(r, r)

    def body(i, carry):
        g0 = i * R
        for r in range(R):
            wait_gathers(r)
            fire_write(g0 + r, r)
        for r in range(R):
            wait_write(r)
            fire_gathers(g0 + R + r, r)
        return carry

    lax.fori_loop(0, STEADY, body, 0)

    g0 = STEADY * R
    for r in range(R):
        wait_gathers(r)
        fire_write(g0 + r, r)
    for r in range(R):
        wait_write(r)


def kernel(plan_ids, weight):
    return _gather_kernel(weight, plan_ids.astype(jnp.int32))


# speedup vs baseline: 2.2583x; 1.1976x over previous
import functools

import jax
import jax.numpy as jnp
from jax import lax
from jax.experimental import pallas as pl
from jax.experimental.pallas import tpu as pltpu
from jax.experimental.pallas import tpu_sc as plsc

NC = 2
NS = 16
NW = NC * NS

VOCAB = 1_000_000
D = 64
DP = 128
N = 16384
K = 50
KP = 56
NP = N // NW
NBUF = 4
G = NP // NBUF
R = 2
STEADY = (G - R) // R


@functools.partial(
    pl.kernel,
    mesh=plsc.VectorSubcoreMesh(core_axis_name="c", subcore_axis_name="s"),
    out_type=jax.ShapeDtypeStruct((N, KP, DP), jnp.float32),
    scratch_types=[
        pltpu.VMEM((NP, K), jnp.int32),
        pltpu.VMEM((R, NBUF, KP, DP), jnp.float32),
        pltpu.SemaphoreType.DMA,
        pltpu.SemaphoreType.DMA,
    ],
    compiler_params=pltpu.CompilerParams(use_tc_tiling_on_sc=False),
)
def _gather_kernel(table_hbm, idx_hbm, out_hbm, idx_v, bufs, gsem, wsem):
    wid = lax.axis_index("s") * NC + lax.axis_index("c")
    i0 = wid * NP

    pltpu.sync_copy(idx_hbm.at[pl.ds(i0, NP)], idx_v)

    def fire_gathers(g, r):
        for b in range(NBUF):
            pltpu.async_copy(
                table_hbm.at[idx_v.at[g * NBUF + b]],
                bufs.at[r, b, pl.ds(0, K)],
                gsem,
            )

    def wait_gathers(r):
        for b in range(NBUF):
            pltpu.make_async_copy(
                table_hbm.at[idx_v.at[0]], bufs.at[r, b, pl.ds(0, K)], gsem
            ).wait()

    def fire_write(g, r):
        pltpu.async_copy(
            bufs.at[r], out_hbm.at[pl.ds(i0 + g * NBUF, NBUF)], wsem
        )

    def wait_write(r):
        pltpu.make_async_copy(
            bufs.at[r], out_hbm.at[pl.ds(i0, NBUF)], wsem
        ).wait()

    for r in range(R):
        fire_gathers(r, r)

    def body(i, carry):
        g0 = i * R
        for r in range(R):
            wait_gathers(r)
            fire_write(g0 + r, r)
        for r in range(R):
            wait_write(r)
            fire_gathers(g0 + R + r, r)
        return carry

    lax.fori_loop(0, STEADY, body, 0)

    g0 = STEADY * R
    for r in range(R):
        wait_gathers(r)
        fire_write(g0 + r, r)
    for r in range(R):
        wait_write(r)


def kernel(plan_ids, weight):
    w2 = jnp.pad(weight, ((0, 0), (0, DP - D)))
    out_padded = _gather_kernel(w2, plan_ids.astype(jnp.int32))
    return out_padded[:, :K, :D]
